# Initial kernel scaffold; baseline (speedup 1.0000x reference)
import jax
import jax.numpy as jnp
from jax.experimental import pallas as pl


def kernel(x, weight, mask):
    raise NotImplementedError("write your pallas kernel here")


# speedup vs baseline: 1.1860x; 1.1860x over previous
import jax
import jax.numpy as jnp
from jax.experimental import pallas as pl
from jax.experimental.pallas import tpu as pltpu


def _pick_tile(n, preferred, multiple):
    if n <= preferred:
        return n
    t = (preferred // multiple) * multiple
    while t >= multiple:
        if n % t == 0:
            return t
        t -= multiple
    return n


def _pooled_head_kernel(x_ref, m_ref, w_ref, *refs):
    out_refs = refs[:-2]
    acc_ref, cnt_ref = refs[-2:]
    s = pl.program_id(1)

    @pl.when(s == 0)
    def _():
        acc_ref[...] = jnp.zeros_like(acc_ref)
        cnt_ref[...] = jnp.zeros_like(cnt_ref)

    x = x_ref[...].astype(jnp.float32)
    v = jnp.where(m_ref[...], 0.0, 1.0)
    acc_ref[...] += jnp.sum(x * v[:, :, None], axis=1)
    cnt_ref[...] += jnp.sum(v, axis=1, keepdims=True)

    @pl.when(s == pl.num_programs(1) - 1)
    def _():
        inv = 1.0 / jnp.maximum(cnt_ref[...], 1.0)
        pooled = acc_ref[...] * inv
        out = jax.lax.dot_general(
            pooled, w_ref[...],
            dimension_numbers=(((1,), (1,)), ((), ())),
            preferred_element_type=jnp.float32,
        )
        for i, oref in enumerate(out_refs):
            oref[...] = out[:, i:i + 1].astype(oref.dtype)


def kernel(x, weight, mask):
    B, S, D = x.shape
    T = weight.shape[0]
    out_dtype = jnp.promote_types(x.dtype, weight.dtype)

    B_tile = _pick_tile(B, 32, 32 if B % 32 == 0 else 8)
    itemsize = jnp.dtype(x.dtype).itemsize
    s_budget = max(128, (16 * 1024 * 1024) // max(1, B_tile * D * itemsize))
    S_tile = _pick_tile(S, s_budget, 128 if S % 128 == 0 else 8)
    grid = (B // B_tile, S // S_tile)

    x_bytes = B_tile * S_tile * D * itemsize
    vmem_limit = int(min(2 * x_bytes + (8 << 20), 100 << 20))

    out = pl.pallas_call(
        _pooled_head_kernel,
        out_shape=[jax.ShapeDtypeStruct((B, 1), out_dtype) for _ in range(T)],
        grid=grid,
        in_specs=[
            pl.BlockSpec((B_tile, S_tile, D), lambda b, s: (b, s, 0)),
            pl.BlockSpec((B_tile, S_tile), lambda b, s: (b, s)),
            pl.BlockSpec((T, D), lambda b, s: (0, 0)),
        ],
        out_specs=[pl.BlockSpec((B_tile, 1), lambda b, s: (b, 0)) for _ in range(T)],
        scratch_shapes=[
            pltpu.VMEM((B_tile, D), jnp.float32),
            pltpu.VMEM((B_tile, 1), jnp.float32),
        ],
        compiler_params=pltpu.CompilerParams(
            dimension_semantics=("parallel", "arbitrary"),
            vmem_limit_bytes=vmem_limit,
        ),
    )(x, mask, weight)

    return {f"t{i}": out[i] for i in range(T)}
